# Initial kernel scaffold; baseline (speedup 1.0000x reference)
import jax
import jax.numpy as jnp
from jax.experimental import pallas as pl


def kernel(ranks, rerank_dba_final, res_top1000_dba, ranks_trans_1000_pre, x_dba):
    raise NotImplementedError("write your pallas kernel here")


# speedup vs baseline: 6.7353x; 6.7353x over previous
import jax
import jax.numpy as jnp
from jax.experimental import pallas as pl
from jax.experimental.pallas import tpu as pltpu

_K = 10


def _rerank_body(pre_smem, pre_row_ref, scores_row_ref, ids_row_ref, x_ref,
                 out_ref):
    M = x_ref.shape[1]
    x = x_ref[0]

    X1 = x_ref[0, pl.ds(pre_smem[0, 0, 0], 1), :]
    for k in range(1, _K):
        X1 = jnp.maximum(X1, x_ref[0, pl.ds(pre_smem[0, 0, k], 1), :])

    xr = x.astype(jnp.bfloat16).astype(jnp.float32)
    X1r = X1.astype(jnp.bfloat16).astype(jnp.float32)
    s_col = jnp.sum(xr * X1r, axis=1, keepdims=True)

    v_row = scores_row_ref[0]
    ids_row = ids_row_ref[0]
    pre_row = pre_row_ref[0]

    iota_r = jax.lax.broadcasted_iota(jnp.int32, (M, M), 1)
    iota_c = jax.lax.broadcasted_iota(jnp.int32, (M, M), 0)
    eid = iota_r == iota_c

    def t_row_to_col(row, zero):
        return jnp.sum(jnp.where(eid, row, zero), axis=1, keepdims=True)

    v_col = t_row_to_col(v_row, 0.0)

    cnt1 = (v_row > v_col) | ((v_row == v_col) & (iota_r < iota_c))
    rank1_col = jnp.sum(cnt1.astype(jnp.int32), axis=1, keepdims=True)

    sorted_v_row = jnp.sum(jnp.where(rank1_col == iota_r, v_col, 0.0),
                           axis=0, keepdims=True)

    s_g_row = jnp.sum(jnp.where(iota_c == pre_row, s_col, 0.0),
                      axis=0, keepdims=True)

    r_row = (sorted_v_row + s_g_row) * 0.5
    r_col = t_row_to_col(r_row, 0.0)

    cnt2 = (r_row > r_col) | ((r_row == r_col) & (iota_r < iota_c))
    rank2_col = jnp.sum(cnt2.astype(jnp.int32), axis=1, keepdims=True)

    ids_col = t_row_to_col(ids_row, 0)
    out_row = jnp.sum(jnp.where(rank2_col == iota_r, ids_col, 0),
                      axis=0, keepdims=True)
    out_ref[0] = out_row


def kernel(ranks, rerank_dba_final, res_top1000_dba, ranks_trans_1000_pre,
           x_dba):
    Q, M = ranks_trans_1000_pre.shape
    D = x_dba.shape[2]
    pre3 = ranks_trans_1000_pre.reshape(Q, 1, M)
    scores3 = res_top1000_dba.reshape(Q, 1, M)
    ids3 = rerank_dba_final.reshape(Q, 1, M)
    out3 = pl.pallas_call(
        _rerank_body,
        grid=(Q,),
        in_specs=[
            pl.BlockSpec((1, 1, M), lambda q: (q, 0, 0),
                         memory_space=pltpu.SMEM),
            pl.BlockSpec((1, 1, M), lambda q: (q, 0, 0)),
            pl.BlockSpec((1, 1, M), lambda q: (q, 0, 0)),
            pl.BlockSpec((1, 1, M), lambda q: (q, 0, 0)),
            pl.BlockSpec((1, M, D), lambda q: (q, 0, 0)),
        ],
        out_specs=pl.BlockSpec((1, 1, M), lambda q: (q, 0, 0)),
        out_shape=jax.ShapeDtypeStruct((Q, 1, M), jnp.int32),
    )(pre3, pre3, scores3, ids3, x_dba)
    reranked_t = out3.reshape(Q, M).T
    return jnp.concatenate([reranked_t, ranks[M:, :]], axis=0)
